# baseline (speedup 1.0000x reference)
import functools

import jax
import jax.numpy as jnp
from jax.experimental import pallas as pl
from jax.experimental.pallas import tpu as pltpu

TAU = 0.07
B, C, H, W = 8, 96, 128, 128
K = 21
HW = H * W
BLK = 2048
NB = HW // BLK


def _proto_body(feat_ref, gt_ref, k0_ref, npos_ref):
    step = pl.program_id(0) * NB + pl.program_id(1)
    mask = (gt_ref[0] == 1).astype(jnp.float32)
    f = feat_ref[0]
    part = jax.lax.dot_general(mask, f, (((1,), (1,)), ((), ())),
                               preferred_element_type=jnp.float32)

    @pl.when(step == 0)
    def _init():
        k0_ref[...] = jnp.zeros_like(k0_ref)
        npos_ref[...] = jnp.zeros_like(npos_ref)

    k0_ref[...] += part
    npos_ref[...] += jnp.sum(mask)


def _loss_body(k0_ref, npos_ref, feat_ref, gt_ref, out_ref, acc_ref):
    step = pl.program_id(0) * NB + pl.program_id(1)

    @pl.when(step == 0)
    def _init():
        acc_ref[0] = 0.0

    k0 = k0_ref[...]
    k0n = k0 / jnp.maximum(
        jnp.sqrt(jnp.sum(k0 * k0, axis=1, keepdims=True)), 1e-12)
    f = feat_ref[0]
    inv_f = 1.0 / jnp.maximum(
        jnp.sqrt(jnp.sum(f * f, axis=0, keepdims=True)), 1e-12)
    s = jax.lax.dot_general(k0n, f, (((1,), (0,)), ((), ())),
                            preferred_element_type=jnp.float32)
    s = s * inv_f / TAU
    denom = jnp.sum(jnp.exp(s), axis=0, keepdims=True)
    mask = (gt_ref[0] == 1).astype(jnp.float32)
    acc_ref[0] += jnp.sum(mask * (s - jnp.log(denom)))

    @pl.when(step == B * NB - 1)
    def _fin():
        out_ref[...] = -(acc_ref[0] / npos_ref[...])


@jax.jit
def kernel(feat, gt):
    featr = feat.reshape(B, C, HW)
    gtr = gt.reshape(B, K, HW)
    k0, npos = pl.pallas_call(
        _proto_body,
        grid=(B, NB),
        in_specs=[
            pl.BlockSpec((1, C, BLK), lambda b, j: (b, 0, j)),
            pl.BlockSpec((1, K, BLK), lambda b, j: (b, 0, j)),
        ],
        out_specs=[
            pl.BlockSpec((K, C), lambda b, j: (0, 0)),
            pl.BlockSpec((1, 1), lambda b, j: (0, 0)),
        ],
        out_shape=[
            jax.ShapeDtypeStruct((K, C), jnp.float32),
            jax.ShapeDtypeStruct((1, 1), jnp.float32),
        ],
    )(featr, gtr)

    loss = pl.pallas_call(
        _loss_body,
        grid=(B, NB),
        in_specs=[
            pl.BlockSpec((K, C), lambda b, j: (0, 0)),
            pl.BlockSpec((1, 1), lambda b, j: (0, 0)),
            pl.BlockSpec((1, C, BLK), lambda b, j: (b, 0, j)),
            pl.BlockSpec((1, K, BLK), lambda b, j: (b, 0, j)),
        ],
        out_specs=pl.BlockSpec((1, 1), lambda b, j: (0, 0)),
        out_shape=jax.ShapeDtypeStruct((1, 1), jnp.float32),
        scratch_shapes=[pltpu.SMEM((1,), jnp.float32)],
    )(k0, npos, featr, gtr)
    return loss.reshape(1)
